# Initial kernel scaffold; baseline (speedup 1.0000x reference)
import jax
import jax.numpy as jnp
from jax.experimental import pallas as pl


def kernel(x, mask, pe_k, pe_v, bases, comp, root, bias, W1, W2, Wp):
    raise NotImplementedError("write your pallas kernel here")


# speedup vs baseline: 15.2904x; 15.2904x over previous
import functools

import jax
import jax.numpy as jnp
import numpy as np
from jax.experimental import pallas as pl

WINDOW = 7
REL_NUM = WINDOW + 2
MAX_LEN = 10
L = 128
D = 300
MLP = 256
TI = 32


@functools.lru_cache(maxsize=None)
def _static_graph(slen: int):
    i = np.arange(slen)[:, None]
    j = np.arange(slen)[None, :]
    rel_adj = np.where(j > i, 1, 0).astype(np.int64)
    d = i - j
    lower = -np.minimum(np.ceil(d / 2.0), float(WINDOW + 1)).astype(np.int64)
    rel_adj = np.where(j < i, lower, rel_adj)
    T = (rel_adj % REL_NUM).astype(np.int64)
    cnt = np.zeros((slen, REL_NUM), dtype=np.float64)
    for r in range(REL_NUM):
        cnt[:, r] = (T == r).sum(axis=0)
    invcnt = 1.0 / np.maximum(cnt, 1.0)
    Mt = np.zeros((REL_NUM, slen, slen), dtype=np.float32)
    for r in range(REL_NUM):
        Mt[r] = ((T == r).T * invcnt[:, r][:, None]).astype(np.float32)
    return jnp.asarray(Mt)


def _stage1(x_ref, mt_ref, bases_ref, comp_ref, root_ref, bias_ref,
            w1a_ref, w1c_ref, s_ref, t_ref):
    xb = x_ref[0]
    a0 = jnp.zeros((L, L), dtype=jnp.float32)
    a1 = jnp.zeros((L, L), dtype=jnp.float32)
    for r in range(REL_NUM):
        a0 = a0 + mt_ref[r] * comp_ref[r:r + 1, 0:1]
        a1 = a1 + mt_ref[r] * comp_ref[r:r + 1, 1:2]
    h0 = jnp.dot(xb, bases_ref[0], preferred_element_type=jnp.float32)
    h1 = jnp.dot(xb, bases_ref[1], preferred_element_type=jnp.float32)
    out = (jnp.dot(a0, h0, preferred_element_type=jnp.float32)
           + jnp.dot(a1, h1, preferred_element_type=jnp.float32)
           + jnp.dot(xb, root_ref[...], preferred_element_type=jnp.float32)
           + bias_ref[...])
    s_ref[0] = jnp.dot(out, w1a_ref[...], preferred_element_type=jnp.float32)
    t_ref[0] = jnp.dot(out, w1c_ref[...], preferred_element_type=jnp.float32)


def _stage2(s_ref, t_ref, pek_ref, pev_ref, w1b_ref, w1d_ref, w2_ref,
            wp_ref, mask_ref, o_ref):
    it = pl.program_id(1)
    base = it * TI
    rtab = (jnp.dot(pek_ref[...], w1b_ref[...], preferred_element_type=jnp.float32)
            + jnp.dot(pev_ref[...], w1d_ref[...], preferred_element_type=jnp.float32))
    f = jax.lax.broadcasted_iota(jnp.int32, (TI * L, 1), 0)
    iv = base + f // L
    jv = f % L
    pm = jnp.clip(iv - jv + 1, 0, MAX_LEN)
    oh = (pm == jax.lax.broadcasted_iota(jnp.int32, (TI * L, MAX_LEN + 1), 1)
          ).astype(jnp.float32)
    rg = jnp.dot(oh, rtab, preferred_element_type=jnp.float32)
    ts = (t_ref[0][:, None, :] + s_ref[0][None, :, :]).reshape(TI * L, MLP)
    h1 = jnp.maximum(ts + rg, 0.0)
    h2 = jnp.maximum(jnp.dot(h1, w2_ref[...], preferred_element_type=jnp.float32), 0.0)
    score = jnp.sum((h2 * wp_ref[...]).reshape(TI, L, MLP), axis=-1)
    o_ref[0] = jax.nn.sigmoid(score) * mask_ref[0]


def kernel(x, mask, pe_k, pe_v, bases, comp, root, bias, W1, W2, Wp):
    B = x.shape[0]
    mt = _static_graph(L)
    w1a = W1[:D]
    w1b = W1[D:D + 100]
    w1c = W1[D + 100:2 * D + 100]
    w1d = W1[2 * D + 100:]
    bias2 = bias.reshape(1, D)
    wp_row = Wp.reshape(1, MLP)

    s, t = pl.pallas_call(
        _stage1,
        grid=(B,),
        in_specs=[
            pl.BlockSpec((1, L, D), lambda b: (b, 0, 0)),
            pl.BlockSpec((REL_NUM, L, L), lambda b: (0, 0, 0)),
            pl.BlockSpec((2, D, D), lambda b: (0, 0, 0)),
            pl.BlockSpec((REL_NUM, 2), lambda b: (0, 0)),
            pl.BlockSpec((D, D), lambda b: (0, 0)),
            pl.BlockSpec((1, D), lambda b: (0, 0)),
            pl.BlockSpec((D, MLP), lambda b: (0, 0)),
            pl.BlockSpec((D, MLP), lambda b: (0, 0)),
        ],
        out_specs=[
            pl.BlockSpec((1, L, MLP), lambda b: (b, 0, 0)),
            pl.BlockSpec((1, L, MLP), lambda b: (b, 0, 0)),
        ],
        out_shape=[
            jax.ShapeDtypeStruct((B, L, MLP), jnp.float32),
            jax.ShapeDtypeStruct((B, L, MLP), jnp.float32),
        ],
    )(x, mt, bases, comp, root, bias2, w1a, w1c)

    out = pl.pallas_call(
        _stage2,
        grid=(B, L // TI),
        in_specs=[
            pl.BlockSpec((1, L, MLP), lambda b, i: (b, 0, 0)),
            pl.BlockSpec((1, TI, MLP), lambda b, i: (b, i, 0)),
            pl.BlockSpec((MAX_LEN + 1, 100), lambda b, i: (0, 0)),
            pl.BlockSpec((MAX_LEN + 1, 100), lambda b, i: (0, 0)),
            pl.BlockSpec((100, MLP), lambda b, i: (0, 0)),
            pl.BlockSpec((100, MLP), lambda b, i: (0, 0)),
            pl.BlockSpec((MLP, MLP), lambda b, i: (0, 0)),
            pl.BlockSpec((1, MLP), lambda b, i: (0, 0)),
            pl.BlockSpec((1, TI, L), lambda b, i: (b, i, 0)),
        ],
        out_specs=pl.BlockSpec((1, TI, L), lambda b, i: (b, i, 0)),
        out_shape=jax.ShapeDtypeStruct((B, L, L), jnp.float32),
    )(s, t, pe_k, pe_v, w1b, w1d, W2, wp_row, mask)
    return out


# speedup vs baseline: 40.1493x; 2.6258x over previous
import functools

import jax
import jax.numpy as jnp
import numpy as np
from jax.experimental import pallas as pl

WINDOW = 7
REL_NUM = WINDOW + 2
MAX_LEN = 10
L = 128
D = 300
MLP = 256
TI = 32


@functools.lru_cache(maxsize=None)
def _static_graph(slen: int):
    i = np.arange(slen)[:, None]
    j = np.arange(slen)[None, :]
    rel_adj = np.where(j > i, 1, 0).astype(np.int64)
    d = i - j
    lower = -np.minimum(np.ceil(d / 2.0), float(WINDOW + 1)).astype(np.int64)
    rel_adj = np.where(j < i, lower, rel_adj)
    T = (rel_adj % REL_NUM).astype(np.int64)
    cnt = np.zeros((slen, REL_NUM), dtype=np.float64)
    for r in range(REL_NUM):
        cnt[:, r] = (T == r).sum(axis=0)
    invcnt = 1.0 / np.maximum(cnt, 1.0)
    Mt = np.zeros((REL_NUM, slen, slen), dtype=np.float32)
    for r in range(REL_NUM):
        Mt[r] = ((T == r).T * invcnt[:, r][:, None]).astype(np.float32)
    return jnp.asarray(Mt)


@functools.lru_cache(maxsize=None)
def _static_g0(ti: int):
    g0 = np.zeros((ti + L, ti * L), dtype=np.float32)
    f = np.arange(ti * L)
    g0[f // L, f] = 1.0
    g0[ti + f % L, f] = 1.0
    return jnp.asarray(g0)


def _stage1(x_ref, mt_ref, bases_ref, comp_ref, root_ref, bias_ref,
            w1at_ref, w1ct_ref, s_ref, t_ref):
    xb = x_ref[0]
    a0 = jnp.zeros((L, L), dtype=jnp.float32)
    a1 = jnp.zeros((L, L), dtype=jnp.float32)
    for r in range(REL_NUM):
        a0 = a0 + mt_ref[r] * comp_ref[r:r + 1, 0:1]
        a1 = a1 + mt_ref[r] * comp_ref[r:r + 1, 1:2]
    h0 = jnp.dot(xb, bases_ref[0], preferred_element_type=jnp.float32)
    h1 = jnp.dot(xb, bases_ref[1], preferred_element_type=jnp.float32)
    out = (jnp.dot(a0, h0, preferred_element_type=jnp.float32)
           + jnp.dot(a1, h1, preferred_element_type=jnp.float32)
           + jnp.dot(xb, root_ref[...], preferred_element_type=jnp.float32)
           + bias_ref[...])
    outT = out.T
    s_ref[0] = jnp.dot(w1at_ref[...], outT, preferred_element_type=jnp.float32)
    t_ref[0] = jnp.dot(w1ct_ref[...], outT, preferred_element_type=jnp.float32)


def _stage2(s_ref, t_ref, g0_ref, pekt_ref, pevt_ref, w1bt_ref, w1dt_ref,
            w2t_ref, wp_ref, mask_ref, o_ref):
    it = pl.program_id(1)
    base = it * TI
    rtab = (jnp.dot(w1bt_ref[...], pekt_ref[...], preferred_element_type=jnp.float32)
            + jnp.dot(w1dt_ref[...], pevt_ref[...], preferred_element_type=jnp.float32))
    sT = s_ref[0]
    tT = t_ref[0, 0]
    f = jax.lax.broadcasted_iota(jnp.int32, (1, TI * L), 1)
    pm = jnp.clip(base + (f >> 7) - (f & (L - 1)) + 1, 0, MAX_LEN)
    kv = jax.lax.broadcasted_iota(jnp.int32, (MAX_LEN + 1, TI * L), 0)
    ohpm = (kv == pm).astype(jnp.float32)
    g = jnp.concatenate([g0_ref[...], ohpm], axis=0)
    w = jnp.concatenate([tT, sT, rtab], axis=1)
    h1 = jnp.maximum(jnp.dot(w, g, preferred_element_type=jnp.float32), 0.0)
    h2 = jnp.maximum(jnp.dot(w2t_ref[...], h1, preferred_element_type=jnp.float32), 0.0)
    wpr = wp_ref[...]
    for ii in range(TI):
        srow = jnp.dot(wpr, h2[:, ii * L:(ii + 1) * L],
                       preferred_element_type=jnp.float32)
        o_ref[0, ii:ii + 1, :] = jax.nn.sigmoid(srow) * mask_ref[0, ii:ii + 1, :]


def kernel(x, mask, pe_k, pe_v, bases, comp, root, bias, W1, W2, Wp):
    B = x.shape[0]
    mt = _static_graph(L)
    w1at = W1[:D].T
    w1bt = W1[D:D + 100].T
    w1ct = W1[D + 100:2 * D + 100].T
    w1dt = W1[2 * D + 100:].T
    bias2 = bias.reshape(1, D)
    wp_row = Wp.reshape(1, MLP)
    pekt = pe_k.T
    pevt = pe_v.T
    w2t = W2.T

    s, t = pl.pallas_call(
        _stage1,
        grid=(B,),
        in_specs=[
            pl.BlockSpec((1, L, D), lambda b: (b, 0, 0)),
            pl.BlockSpec((REL_NUM, L, L), lambda b: (0, 0, 0)),
            pl.BlockSpec((2, D, D), lambda b: (0, 0, 0)),
            pl.BlockSpec((REL_NUM, 2), lambda b: (0, 0)),
            pl.BlockSpec((D, D), lambda b: (0, 0)),
            pl.BlockSpec((1, D), lambda b: (0, 0)),
            pl.BlockSpec((MLP, D), lambda b: (0, 0)),
            pl.BlockSpec((MLP, D), lambda b: (0, 0)),
        ],
        out_specs=[
            pl.BlockSpec((1, MLP, L), lambda b: (b, 0, 0)),
            pl.BlockSpec((1, MLP, L), lambda b: (b, 0, 0)),
        ],
        out_shape=[
            jax.ShapeDtypeStruct((B, MLP, L), jnp.float32),
            jax.ShapeDtypeStruct((B, MLP, L), jnp.float32),
        ],
    )(x, mt, bases, comp, root, bias2, w1at, w1ct)

    t4 = t.reshape(B, MLP, L // TI, TI).transpose(0, 2, 1, 3)

    out = pl.pallas_call(
        _stage2,
        grid=(B, L // TI),
        in_specs=[
            pl.BlockSpec((1, MLP, L), lambda b, i: (b, 0, 0)),
            pl.BlockSpec((1, 1, MLP, TI), lambda b, i: (b, i, 0, 0)),
            pl.BlockSpec((TI + L, TI * L), lambda b, i: (0, 0)),
            pl.BlockSpec((100, MAX_LEN + 1), lambda b, i: (0, 0)),
            pl.BlockSpec((100, MAX_LEN + 1), lambda b, i: (0, 0)),
            pl.BlockSpec((MLP, 100), lambda b, i: (0, 0)),
            pl.BlockSpec((MLP, 100), lambda b, i: (0, 0)),
            pl.BlockSpec((MLP, MLP), lambda b, i: (0, 0)),
            pl.BlockSpec((1, MLP), lambda b, i: (0, 0)),
            pl.BlockSpec((1, TI, L), lambda b, i: (b, i, 0)),
        ],
        out_specs=pl.BlockSpec((1, TI, L), lambda b, i: (b, i, 0)),
        out_shape=jax.ShapeDtypeStruct((B, L, L), jnp.float32),
    )(s, t4, _static_g0(TI), pekt, pevt, w1bt, w1dt, w2t, wp_row, mask)
    return out


# speedup vs baseline: 45.9074x; 1.1434x over previous
import functools

import jax
import jax.numpy as jnp
import numpy as np
from jax.experimental import pallas as pl

WINDOW = 7
REL_NUM = WINDOW + 2
MAX_LEN = 10
L = 128
D = 300
MLP = 256
TI = 64
NI = L // TI


@functools.lru_cache(maxsize=None)
def _static_graph(slen: int):
    i = np.arange(slen)[:, None]
    j = np.arange(slen)[None, :]
    rel_adj = np.where(j > i, 1, 0).astype(np.int64)
    d = i - j
    lower = -np.minimum(np.ceil(d / 2.0), float(WINDOW + 1)).astype(np.int64)
    rel_adj = np.where(j < i, lower, rel_adj)
    T = (rel_adj % REL_NUM).astype(np.int64)
    cnt = np.zeros((slen, REL_NUM), dtype=np.float64)
    for r in range(REL_NUM):
        cnt[:, r] = (T == r).sum(axis=0)
    invcnt = 1.0 / np.maximum(cnt, 1.0)
    Mt = np.zeros((REL_NUM, slen, slen), dtype=np.float32)
    for r in range(REL_NUM):
        Mt[r] = ((T == r).T * invcnt[:, r][:, None]).astype(np.float32)
    return jnp.asarray(Mt)


@functools.lru_cache(maxsize=None)
def _static_g0(ti: int):
    g0 = np.zeros((ti + L, ti * L), dtype=np.float32)
    f = np.arange(ti * L)
    g0[f // L, f] = 1.0
    g0[ti + f % L, f] = 1.0
    return jnp.asarray(g0, dtype=jnp.bfloat16)


def _stage1(x_ref, mt_ref, bases_ref, comp_ref, root_ref, bias_ref,
            w1at_ref, w1ct_ref, s_ref, t_ref):
    xb = x_ref[0]
    a0 = jnp.zeros((L, L), dtype=jnp.float32)
    a1 = jnp.zeros((L, L), dtype=jnp.float32)
    for r in range(REL_NUM):
        a0 = a0 + mt_ref[r] * comp_ref[r:r + 1, 0:1]
        a1 = a1 + mt_ref[r] * comp_ref[r:r + 1, 1:2]
    h0 = jnp.dot(xb, bases_ref[0], preferred_element_type=jnp.float32)
    h1 = jnp.dot(xb, bases_ref[1], preferred_element_type=jnp.float32)
    out = (jnp.dot(a0, h0, preferred_element_type=jnp.float32)
           + jnp.dot(a1, h1, preferred_element_type=jnp.float32)
           + jnp.dot(xb, root_ref[...], preferred_element_type=jnp.float32)
           + bias_ref[...])
    outT = out.T
    s_ref[0] = jnp.dot(w1at_ref[...], outT, preferred_element_type=jnp.float32)
    tT = jnp.dot(w1ct_ref[...], outT, preferred_element_type=jnp.float32)
    for n in range(NI):
        t_ref[0, n] = tT[:, n * TI:(n + 1) * TI]


def _stage2(s_ref, t_ref, g0_ref, pekt_ref, pevt_ref, w1bt_ref, w1dt_ref,
            w2t_ref, wp_ref, mask_ref, o_ref):
    it = pl.program_id(1)
    base = it * TI
    rtab = (jnp.dot(w1bt_ref[...], pekt_ref[...], preferred_element_type=jnp.float32)
            + jnp.dot(w1dt_ref[...], pevt_ref[...], preferred_element_type=jnp.float32))
    sT = s_ref[0]
    tT = t_ref[0, 0]
    f = jax.lax.broadcasted_iota(jnp.int32, (1, TI * L), 1)
    pm = jnp.clip(base + (f >> 7) - (f & (L - 1)) + 1, 0, MAX_LEN)
    kv = jax.lax.broadcasted_iota(jnp.int32, (MAX_LEN + 1, TI * L), 0)
    ohpm = (kv == pm).astype(jnp.bfloat16)
    g = jnp.concatenate([g0_ref[...], ohpm], axis=0)
    w = jnp.concatenate([tT, sT, rtab], axis=1).astype(jnp.bfloat16)
    h1 = jnp.maximum(jnp.dot(w, g, preferred_element_type=jnp.float32), 0.0)
    h1 = h1.astype(jnp.bfloat16)
    h2 = jnp.maximum(jnp.dot(w2t_ref[...], h1,
                             preferred_element_type=jnp.float32), 0.0)
    wpr = wp_ref[...]
    for ii in range(TI):
        srow = jnp.dot(wpr, h2[:, ii * L:(ii + 1) * L],
                       preferred_element_type=jnp.float32)
        o_ref[0, ii:ii + 1, :] = jax.nn.sigmoid(srow) * mask_ref[0, ii:ii + 1, :]


def kernel(x, mask, pe_k, pe_v, bases, comp, root, bias, W1, W2, Wp):
    B = x.shape[0]
    mt = _static_graph(L)
    w1at = W1[:D].T
    w1bt = W1[D:D + 100].T
    w1ct = W1[D + 100:2 * D + 100].T
    w1dt = W1[2 * D + 100:].T
    bias2 = bias.reshape(1, D)
    wp_row = Wp.reshape(1, MLP)
    pekt = pe_k.T
    pevt = pe_v.T
    w2t = W2.T.astype(jnp.bfloat16)

    s, t = pl.pallas_call(
        _stage1,
        grid=(B,),
        in_specs=[
            pl.BlockSpec((1, L, D), lambda b: (b, 0, 0)),
            pl.BlockSpec((REL_NUM, L, L), lambda b: (0, 0, 0)),
            pl.BlockSpec((2, D, D), lambda b: (0, 0, 0)),
            pl.BlockSpec((REL_NUM, 2), lambda b: (0, 0)),
            pl.BlockSpec((D, D), lambda b: (0, 0)),
            pl.BlockSpec((1, D), lambda b: (0, 0)),
            pl.BlockSpec((MLP, D), lambda b: (0, 0)),
            pl.BlockSpec((MLP, D), lambda b: (0, 0)),
        ],
        out_specs=[
            pl.BlockSpec((1, MLP, L), lambda b: (b, 0, 0)),
            pl.BlockSpec((1, NI, MLP, TI), lambda b: (b, 0, 0, 0)),
        ],
        out_shape=[
            jax.ShapeDtypeStruct((B, MLP, L), jnp.float32),
            jax.ShapeDtypeStruct((B, NI, MLP, TI), jnp.float32),
        ],
    )(x, mt, bases, comp, root, bias2, w1at, w1ct)

    out = pl.pallas_call(
        _stage2,
        grid=(B, L // TI),
        in_specs=[
            pl.BlockSpec((1, MLP, L), lambda b, i: (b, 0, 0)),
            pl.BlockSpec((1, 1, MLP, TI), lambda b, i: (b, i, 0, 0)),
            pl.BlockSpec((TI + L, TI * L), lambda b, i: (0, 0)),
            pl.BlockSpec((100, MAX_LEN + 1), lambda b, i: (0, 0)),
            pl.BlockSpec((100, MAX_LEN + 1), lambda b, i: (0, 0)),
            pl.BlockSpec((MLP, 100), lambda b, i: (0, 0)),
            pl.BlockSpec((MLP, 100), lambda b, i: (0, 0)),
            pl.BlockSpec((MLP, MLP), lambda b, i: (0, 0)),
            pl.BlockSpec((1, MLP), lambda b, i: (0, 0)),
            pl.BlockSpec((1, TI, L), lambda b, i: (b, i, 0)),
        ],
        out_specs=pl.BlockSpec((1, TI, L), lambda b, i: (b, i, 0)),
        out_shape=jax.ShapeDtypeStruct((B, L, L), jnp.float32),
    )(s, t, _static_g0(TI), pekt, pevt, w1bt, w1dt, w2t, wp_row, mask)
    return out
